# baseline (speedup 1.0000x reference)
import functools

import jax
import jax.numpy as jnp
from jax import lax
from jax.experimental import pallas as pl
from jax.experimental.pallas import tpu as pltpu
from jax.experimental.pallas import tpu_sc as plsc

VOCAB = 1000000
MAXLEN = 200
EMBED_DIM = 64
BATCH = 4096
BL = BATCH * MAXLEN
NC, NS, LANES = 2, 16, 16
NW = NC * NS
PER_W = BL // NW
C = MAXLEN
NCHUNK = PER_W // C
NBUF = 4
LOOKAHEAD = 2
G0 = 128
G1 = C - G0
NQ = EMBED_DIM // LANES


def _body(xf, tok_tab, pos_tab, out, idx_all, pos_all, tok, sem_g, sem_w):
  wid = lax.axis_index("s") * NC + lax.axis_index("c")
  base = wid * PER_W
  pltpu.sync_copy(xf.at[pl.ds(base, PER_W)], idx_all)
  pltpu.sync_copy(pos_tab, pos_all)

  def prep(n, k):
    loc = n * C

    @pl.when(n >= NBUF)
    def _():
      pltpu.make_async_copy(tok[k], out.at[pl.ds(0, C)], sem_w[k]).wait()

    pltpu.async_copy(tok_tab.at[idx_all.at[pl.ds(loc, G0)]],
                     tok[k].at[pl.ds(0, G0)], sem_g[k])
    pltpu.async_copy(tok_tab.at[idx_all.at[pl.ds(loc + G0, G1)]],
                     tok[k].at[pl.ds(G0, G1)], sem_g[k])

  def step(n, k):
    @pl.when(n + LOOKAHEAD < NCHUNK)
    def _():
      prep(n + LOOKAHEAD, (k + LOOKAHEAD) % NBUF)

    loc = n * C

    mn = idx_all[pl.ds(loc, LANES)]
    for w in range(1, (C + LANES - 1) // LANES):
      o = min(w * LANES, C - LANES)
      mn = jnp.minimum(mn, idx_all[pl.ds(loc + o, LANES)])
    any_masked = lax.reduce_min(mn, (0,)) == 0

    pltpu.make_async_copy(tok_tab.at[pl.ds(0, G0)], tok[k].at[pl.ds(0, G0)],
                          sem_g[k]).wait()
    pltpu.make_async_copy(tok_tab.at[pl.ds(0, G1)], tok[k].at[pl.ds(G0, G1)],
                          sem_g[k]).wait()

    @pl.loop(0, C, unroll=8)
    def _add(j):
      for q in range(NQ):
        s = pl.ds(q * LANES, LANES)
        plsc.addupdate(tok[k].at[j, s], pos_all[j + 1, s])

    @pl.when(any_masked)
    def _():
      @pl.loop(0, C)
      def _fix(j):
        xi = plsc.load_gather(idx_all, [jnp.full((LANES,), loc + j,
                                                 jnp.int32)])
        delta = jnp.where(xi == 0, 1.0, 0.0)
        for q in range(NQ):
          s = pl.ds(q * LANES, LANES)
          plsc.addupdate(tok[k].at[j, s],
                         delta * (pos_all[0, s] - pos_all[j + 1, s]))

    pltpu.async_copy(tok[k], out.at[pl.ds(base + loc, C)], sem_w[k])

  for p in range(LOOKAHEAD):
    prep(p, p)

  @pl.loop(0, NCHUNK, step=NBUF)
  def _chunks(ci):
    for k in range(NBUF):
      step(ci + k, k)

  for k in range(NBUF):
    pltpu.make_async_copy(tok[k], out.at[pl.ds(0, C)], sem_w[k]).wait()


def _tc_busy(x_ref, o_ref):
  def body(i, acc):
    return acc + x_ref[0:8, :]

  acc = lax.fori_loop(0, 150000, body, jnp.zeros((8, EMBED_DIM),
                                                 jnp.float32))
  o_ref[...] = acc * 1e-45


@functools.partial(jax.jit, donate_argnums=())
def kernel(x, token_table, pos_table):
  mesh = plsc.VectorSubcoreMesh(core_axis_name="c", subcore_axis_name="s")
  run = pl.kernel(
      _body,
      out_type=jax.ShapeDtypeStruct((BL, EMBED_DIM), jnp.float32),
      mesh=mesh,
      scratch_types=[
          pltpu.VMEM((PER_W,), jnp.int32),
          pltpu.VMEM((MAXLEN + 1, EMBED_DIM), jnp.float32),
          [pltpu.VMEM((C, EMBED_DIM), jnp.float32) for _ in range(NBUF)],
          [pltpu.SemaphoreType.DMA for _ in range(NBUF)],
          [pltpu.SemaphoreType.DMA for _ in range(NBUF)],
      ],
      compiler_params=pltpu.CompilerParams(use_tc_tiling_on_sc=False,
                                           needs_layout_passes=False),
  )
  out = run(x.reshape(BL), token_table, pos_table)
  busy = pl.pallas_call(
      _tc_busy,
      out_shape=jax.ShapeDtypeStruct((8, EMBED_DIM), jnp.float32),
  )(pos_table)
  out = out.at[0:8, :].add(busy)
  return out.reshape(BATCH, MAXLEN, EMBED_DIM)


# speedup vs baseline: 1.5084x; 1.5084x over previous
import functools

import jax
import jax.numpy as jnp
from jax import lax
from jax.experimental import pallas as pl
from jax.experimental.pallas import tpu as pltpu
from jax.experimental.pallas import tpu_sc as plsc

VOCAB = 1000000
MAXLEN = 200
EMBED_DIM = 64
BATCH = 4096
BL = BATCH * MAXLEN
NC, NS, LANES = 2, 16, 16
NW = NC * NS
PER_W = BL // NW
C = MAXLEN
NCHUNK = PER_W // C
NBUF = 4
LOOKAHEAD = 3
G0 = 128
G1 = C - G0
NQ = EMBED_DIM // LANES


def _body(xf, tok_tab, pos_tab, out, idx_all, pos_all, tok, sem_g, sem_w):
  wid = lax.axis_index("s") * NC + lax.axis_index("c")
  base = wid * PER_W
  pltpu.sync_copy(xf.at[pl.ds(base, PER_W)], idx_all)
  pltpu.sync_copy(pos_tab, pos_all)

  def prep(n, k):
    loc = n * C

    @pl.when(n >= NBUF)
    def _():
      pltpu.make_async_copy(tok[k], out.at[pl.ds(0, C)], sem_w[k]).wait()

    pltpu.async_copy(tok_tab.at[idx_all.at[pl.ds(loc, G0)]],
                     tok[k].at[pl.ds(0, G0)], sem_g[k])
    pltpu.async_copy(tok_tab.at[idx_all.at[pl.ds(loc + G0, G1)]],
                     tok[k].at[pl.ds(G0, G1)], sem_g[k])

  def step(n, k):
    @pl.when(n + LOOKAHEAD < NCHUNK)
    def _():
      prep(n + LOOKAHEAD, (k + LOOKAHEAD) % NBUF)

    loc = n * C

    mn = idx_all[pl.ds(loc, LANES)]
    for w in range(1, (C + LANES - 1) // LANES):
      o = min(w * LANES, C - LANES)
      mn = jnp.minimum(mn, idx_all[pl.ds(loc + o, LANES)])
    any_masked = lax.reduce_min(mn, (0,)) == 0

    pltpu.make_async_copy(tok_tab.at[pl.ds(0, G0)], tok[k].at[pl.ds(0, G0)],
                          sem_g[k]).wait()
    pltpu.make_async_copy(tok_tab.at[pl.ds(0, G1)], tok[k].at[pl.ds(G0, G1)],
                          sem_g[k]).wait()

    @pl.loop(0, C, unroll=8)
    def _add(j):
      for q in range(NQ):
        s = pl.ds(q * LANES, LANES)
        plsc.addupdate(tok[k].at[j, s], pos_all[j + 1, s])

    @pl.when(any_masked)
    def _():
      @pl.loop(0, C)
      def _fix(j):
        xi = plsc.load_gather(idx_all, [jnp.full((LANES,), loc + j,
                                                 jnp.int32)])
        delta = jnp.where(xi == 0, 1.0, 0.0)
        for q in range(NQ):
          s = pl.ds(q * LANES, LANES)
          plsc.addupdate(tok[k].at[j, s],
                         delta * (pos_all[0, s] - pos_all[j + 1, s]))

    pltpu.async_copy(tok[k], out.at[pl.ds(base + loc, C)], sem_w[k])

  for p in range(LOOKAHEAD):
    prep(p, p)

  @pl.loop(0, NCHUNK, step=NBUF)
  def _chunks(ci):
    for k in range(NBUF):
      step(ci + k, k)

  for k in range(NBUF):
    pltpu.make_async_copy(tok[k], out.at[pl.ds(0, C)], sem_w[k]).wait()


@functools.partial(jax.jit, donate_argnums=())
def kernel(x, token_table, pos_table):
  mesh = plsc.VectorSubcoreMesh(core_axis_name="c", subcore_axis_name="s")
  run = pl.kernel(
      _body,
      out_type=jax.ShapeDtypeStruct((BL, EMBED_DIM), jnp.float32),
      mesh=mesh,
      scratch_types=[
          pltpu.VMEM((PER_W,), jnp.int32),
          pltpu.VMEM((MAXLEN + 1, EMBED_DIM), jnp.float32),
          [pltpu.VMEM((C, EMBED_DIM), jnp.float32) for _ in range(NBUF)],
          [pltpu.SemaphoreType.DMA for _ in range(NBUF)],
          [pltpu.SemaphoreType.DMA for _ in range(NBUF)],
      ],
      compiler_params=pltpu.CompilerParams(use_tc_tiling_on_sc=False,
                                           needs_layout_passes=False),
  )
  out = run(x.reshape(BL), token_table, pos_table)
  return out.reshape(BATCH, MAXLEN, EMBED_DIM)


# speedup vs baseline: 1.5716x; 1.0419x over previous
import functools

import jax
import jax.numpy as jnp
from jax import lax
from jax.experimental import pallas as pl
from jax.experimental.pallas import tpu as pltpu
from jax.experimental.pallas import tpu_sc as plsc

VOCAB = 1000000
MAXLEN = 200
EMBED_DIM = 64
BATCH = 4096
BL = BATCH * MAXLEN
NC, NS, LANES = 2, 16, 16
NW = NC * NS
PER_W = BL // NW
C = MAXLEN
NCHUNK = PER_W // C
NBUF = 4
LOOKAHEAD = 2
G0 = 128
G1 = C - G0
NQ = EMBED_DIM // LANES


def _body(xf, tok_tab, pos_tab, out, idx_all, pos_all, tok, sem_g, sem_w):
  wid = lax.axis_index("s") * NC + lax.axis_index("c")
  base = wid * PER_W
  pltpu.sync_copy(xf.at[pl.ds(base, PER_W)], idx_all)
  pltpu.sync_copy(pos_tab, pos_all)

  def prep(n, k):
    loc = n * C

    @pl.when(n >= NBUF)
    def _():
      pltpu.make_async_copy(tok[k], out.at[pl.ds(0, C)], sem_w[k]).wait()

    pltpu.async_copy(tok_tab.at[idx_all.at[pl.ds(loc, G0)]],
                     tok[k].at[pl.ds(0, G0)], sem_g[k])
    pltpu.async_copy(tok_tab.at[idx_all.at[pl.ds(loc + G0, G1)]],
                     tok[k].at[pl.ds(G0, G1)], sem_g[k])

  def step(n, k):
    @pl.when(n + LOOKAHEAD < NCHUNK)
    def _():
      prep(n + LOOKAHEAD, (k + LOOKAHEAD) % NBUF)

    loc = n * C

    mn = idx_all[pl.ds(loc, LANES)]
    for w in range(1, (C + LANES - 1) // LANES):
      o = min(w * LANES, C - LANES)
      mn = jnp.minimum(mn, idx_all[pl.ds(loc + o, LANES)])
    any_masked = lax.reduce_min(mn, (0,)) == 0

    pltpu.make_async_copy(tok_tab.at[pl.ds(0, G0)], tok[k].at[pl.ds(0, G0)],
                          sem_g[k]).wait()
    pltpu.make_async_copy(tok_tab.at[pl.ds(0, G1)], tok[k].at[pl.ds(G0, G1)],
                          sem_g[k]).wait()

    @pl.loop(0, C, unroll=8)
    def _add(j):
      for q in range(NQ):
        s = pl.ds(q * LANES, LANES)
        plsc.addupdate(tok[k].at[j, s], pos_all[j + 1, s])

    @pl.when(any_masked)
    def _():
      @pl.loop(0, C)
      def _fix(j):
        xi = plsc.load_gather(idx_all, [jnp.full((LANES,), loc + j,
                                                 jnp.int32)])
        delta = jnp.where(xi == 0, 1.0, 0.0)
        for q in range(NQ):
          s = pl.ds(q * LANES, LANES)
          plsc.addupdate(tok[k].at[j, s],
                         delta * (pos_all[0, s] - pos_all[j + 1, s]))

    pltpu.async_copy(tok[k], out.at[pl.ds(base + loc, C)], sem_w[k])

  for p in range(LOOKAHEAD):
    prep(p, p)

  @pl.loop(0, NCHUNK, step=NBUF)
  def _chunks(ci):
    for k in range(NBUF):
      step(ci + k, k)

  for k in range(NBUF):
    pltpu.make_async_copy(tok[k], out.at[pl.ds(0, C)], sem_w[k]).wait()


@functools.partial(jax.jit, donate_argnums=())
def kernel(x, token_table, pos_table):
  mesh = plsc.VectorSubcoreMesh(core_axis_name="c", subcore_axis_name="s")
  run = pl.kernel(
      _body,
      out_type=jax.ShapeDtypeStruct((BL, EMBED_DIM), jnp.float32),
      mesh=mesh,
      scratch_types=[
          pltpu.VMEM((PER_W,), jnp.int32),
          pltpu.VMEM((MAXLEN + 1, EMBED_DIM), jnp.float32),
          [pltpu.VMEM((C, EMBED_DIM), jnp.float32) for _ in range(NBUF)],
          [pltpu.SemaphoreType.DMA for _ in range(NBUF)],
          [pltpu.SemaphoreType.DMA for _ in range(NBUF)],
      ],
      compiler_params=pltpu.CompilerParams(use_tc_tiling_on_sc=False,
                                           needs_layout_passes=False),
  )
  out = run(x.reshape(BL), token_table, pos_table)
  return out.reshape(BATCH, MAXLEN, EMBED_DIM)
